# baseline (speedup 1.0000x reference)
import functools
import math

import jax
import jax.numpy as jnp
from jax import lax
from jax.experimental import pallas as pl
from jax.experimental.pallas import tpu as pltpu
from jax.experimental.pallas import tpu_sc as plsc

_VOCAB = 1000
_D = 128
_BATCH = 4096
_HIST = 50
_NC = 2
_NS = 16
_NW = _NC * _NS
_BPW = _BATCH // _NW
_SCALE = math.sqrt(float(_D))

_SROWS = 64
_SLAST = _VOCAB - (_NS - 1) * _SROWS

_NB = 5
_NGRP = _HIST // _NB

_MESH = plsc.VectorSubcoreMesh(core_axis_name="c", subcore_axis_name="s")
_PARAMS = pltpu.CompilerParams(use_tc_tiling_on_sc=True)


_NHBM = 2


@functools.partial(
    pl.kernel,
    out_type=(jax.ShapeDtypeStruct((_HIST, _BATCH, _D), jnp.float32),
              jax.ShapeDtypeStruct((_VOCAB, _D), jnp.float32)),
    mesh=_MESH,
    compiler_params=_PARAMS,
    scratch_types=[
        pltpu.VMEM((_HIST, _BPW), jnp.int32),
        pltpu.VMEM((_NB, _BPW, _D), jnp.float32),
        pltpu.VMEM_SHARED((_VOCAB, _D), jnp.float32),
        pltpu.SemaphoreType.DMA,
        [pltpu.SemaphoreType.DMA] * _NB,
        [pltpu.SemaphoreType.DMA] * _NB,
    ],
)
def _embed(lut_hbm, idx_hbm, out_hbm, tbl_hbm, idx_v, rows_v, table_sh, isem,
           gsems, ssems):
    sid = lax.axis_index("s")
    wid = sid * _NC + lax.axis_index("c")
    out_base = wid * _BPW

    idx_desc = pltpu.async_copy(idx_hbm.at[wid], idx_v, isem)

    def scale_slice(rows, base):
        stage = rows_v.at[0, pl.ds(0, rows)]
        pltpu.sync_copy(lut_hbm.at[pl.ds(base, rows)], stage)

        def body(r, carry):
            for c in range(_D // 16):
                sl = pl.ds(c * 16, 16)
                stage[r, sl] = stage[r, sl] * _SCALE
            return carry

        lax.fori_loop(0, rows, body, 0)
        pltpu.sync_copy(stage, table_sh.at[pl.ds(base, rows)])
        pltpu.sync_copy(stage, tbl_hbm.at[pl.ds(base, rows)])

    @pl.when(sid < _NS - 1)
    def _():
        scale_slice(_SROWS, sid * _SROWS)

    @pl.when(sid == _NS - 1)
    def _():
        scale_slice(_SLAST, (_NS - 1) * _SROWS)

    idx_desc.wait()
    plsc.subcore_barrier()

    def gather_start(h, b):
        src = tbl_hbm if b < _NHBM else table_sh
        pltpu.async_copy(src.at[idx_v.at[h]], rows_v.at[b], gsems[b])

    def gather_wait(h, b):
        src = tbl_hbm if b < _NHBM else table_sh
        pltpu.make_async_copy(src.at[idx_v.at[h]], rows_v.at[b],
                              gsems[b]).wait()

    def store_start(h, b):
        return pltpu.async_copy(
            rows_v.at[b], out_hbm.at[h, pl.ds(out_base, _BPW)], ssems[b])

    for b in range(_NB):
        gather_start(b, b)

    def body(g, carry):
        base = g * _NB
        descs = []
        for b in range(_NB):
            gather_wait(base + b, b)
            descs.append(store_start(base + b, b))
        for b in range(_NB):
            descs[b].wait()
            gather_start(base + _NB + b, b)
        return carry

    lax.fori_loop(0, _NGRP - 1, body, 0)

    base = (_NGRP - 1) * _NB
    descs = []
    for b in range(_NB):
        gather_wait(base + b, b)
        descs.append(store_start(base + b, b))
    for b in range(_NB):
        descs[b].wait()


def kernel(x, lut):
    idx_t = x.reshape(_NW, _BPW, _HIST).transpose(0, 2, 1)
    out_t, _ = _embed(lut, idx_t)
    return out_t.transpose(1, 0, 2)


# speedup vs baseline: 1.4362x; 1.4362x over previous
import functools
import math

import jax
import jax.numpy as jnp
from jax import lax
from jax.experimental import pallas as pl
from jax.experimental.pallas import tpu as pltpu
from jax.experimental.pallas import tpu_sc as plsc

_VOCAB = 1000
_D = 128
_BATCH = 4096
_HIST = 50
_NC = 2
_NS = 16
_NW = _NC * _NS
_BPW = _BATCH // _NW
_SCALE = math.sqrt(float(_D))

_SROWS = 64
_SLAST = _VOCAB - (_NS - 1) * _SROWS

_HALF = _BPW // 2
_NCHUNK = _HIST * 2
_NB = 5
_NPAIR = _NCHUNK // (2 * _NB)

_MESH = plsc.VectorSubcoreMesh(core_axis_name="c", subcore_axis_name="s")
_PARAMS = pltpu.CompilerParams(use_tc_tiling_on_sc=True)


@functools.partial(
    pl.kernel,
    out_type=jax.ShapeDtypeStruct((_HIST, _BATCH, _D), jnp.float32),
    mesh=_MESH,
    compiler_params=_PARAMS,
    scratch_types=[
        pltpu.VMEM((_HIST, _BPW), jnp.int32),
        pltpu.VMEM((2 * _NB, _HALF, _D), jnp.float32),
        pltpu.VMEM_SHARED((_VOCAB, _D), jnp.float32),
        pltpu.SemaphoreType.DMA,
        [pltpu.SemaphoreType.DMA] * (2 * _NB),
        [pltpu.SemaphoreType.DMA] * (2 * _NB),
    ],
)
def _embed(lut_hbm, idx_hbm, out_hbm, idx_v, rows_v, table_sh, isem,
           gsems, ssems):
    sid = lax.axis_index("s")
    wid = sid * _NC + lax.axis_index("c")
    out_base = wid * _BPW

    idx_desc = pltpu.async_copy(idx_hbm.at[wid], idx_v, isem)

    def scale_slice(rows, base):
        stage = rows_v.at[0, pl.ds(0, rows)]
        pltpu.sync_copy(lut_hbm.at[pl.ds(base, rows)], stage)

        def body(r, carry):
            for c in range(_D // 16):
                sl = pl.ds(c * 16, 16)
                stage[r, sl] = stage[r, sl] * _SCALE
            return carry

        lax.fori_loop(0, rows, body, 0)
        pltpu.sync_copy(stage, table_sh.at[pl.ds(base, rows)])

    @pl.when(sid < _NS - 1)
    def _():
        scale_slice(_SROWS, sid * _SROWS)

    @pl.when(sid == _NS - 1)
    def _():
        scale_slice(_SLAST, (_NS - 1) * _SROWS)

    idx_desc.wait()
    plsc.subcore_barrier()

    def idx_ref(ck):
        return idx_v.at[ck // 2, pl.ds((ck % 2) * _HALF, _HALF)]

    def gather_start(ck, b):
        pltpu.async_copy(table_sh.at[idx_ref(ck)], rows_v.at[b], gsems[b])

    def gather_wait(ck, b):
        pltpu.make_async_copy(table_sh.at[idx_ref(ck)], rows_v.at[b],
                              gsems[b]).wait()

    def store_start(ck, b):
        dst = out_hbm.at[ck // 2,
                         pl.ds(out_base + (ck % 2) * _HALF, _HALF)]
        return pltpu.async_copy(rows_v.at[b], dst, ssems[b])

    for b in range(2 * _NB):
        gather_start(b, b)

    def body(g, refill):
        base = g * 2 * _NB
        descs = []
        for b in range(2 * _NB):
            gather_wait(base + b, b)
            descs.append(store_start(base + b, b))
        for b in range(2 * _NB):
            descs[b].wait()
            if refill:
                gather_start(base + 2 * _NB + b, b)

    def loop_body(g, carry):
        body(g, True)
        return carry

    lax.fori_loop(0, _NPAIR - 1, loop_body, 0)
    body(_NPAIR - 1, False)


def kernel(x, lut):
    idx_t = x.reshape(_NW, _BPW, _HIST).transpose(0, 2, 1)
    out_t = _embed(lut, idx_t)
    return out_t.transpose(1, 0, 2)
